# baseline (speedup 1.0000x reference)
import functools

import jax
import jax.numpy as jnp
from jax import lax
from jax.experimental import pallas as pl
from jax.experimental.pallas import tpu as pltpu
from jax.experimental.pallas import tpu_sc as plsc

S_SC = 4096
BS = 2048


def _sc_part(inputs, table):
    B, S, D = inputs.shape
    info = plsc.get_sparse_core_info()
    NC, NS, L = info.num_cores, info.num_subcores, info.num_lanes
    NW = NC * NS
    SPW = S_SC // NW
    CS = 32
    NCHUNK = SPW // CS
    NVC = D // L

    mesh = plsc.VectorSubcoreMesh(core_axis_name="c", subcore_axis_name="s")

    @functools.partial(
        pl.kernel,
        mesh=mesh,
        out_type=jax.ShapeDtypeStruct((1, S_SC, D), jnp.float32),
        scratch_types=[
            pltpu.VMEM((CS, D), jnp.float32),
            pltpu.VMEM((CS, D), jnp.float32),
        ],
    )
    def k(x_hbm, t_hbm, o_hbm, tbuf, xbuf):
        wid = lax.axis_index("s") * NC + lax.axis_index("c")
        base = wid * SPW

        def chunk_body(c, carry):
            s0 = base + c * CS
            pltpu.sync_copy(t_hbm.at[pl.ds(s0, CS)], tbuf)
            pltpu.sync_copy(x_hbm.at[0, pl.ds(s0, CS)], xbuf)

            def row_body(r, carry2):
                for cc in range(NVC):
                    sl = pl.ds(cc * L, L)
                    xbuf[r, sl] = xbuf[r, sl] + tbuf[r, sl]
                return carry2

            lax.fori_loop(0, CS, row_body, 0)
            pltpu.sync_copy(xbuf, o_hbm.at[0, pl.ds(s0, CS)])
            return carry

        lax.fori_loop(0, NCHUNK, chunk_body, 0)

    return k(inputs, table)


def _tc_part(inputs, table):
    B, S, D = inputs.shape
    NSB = S // BS
    NBLK = B * NSB - S_SC // BS

    def body(x_ref, t_ref, o_ref):
        o_ref[...] = x_ref[...] + t_ref[...]

    skip = S_SC // BS

    def lin(i):
        j = i + 1
        for k in range(1, skip):
            j = j + jnp.where(i >= k * B - 1, 1, 0)
        return j

    return pl.pallas_call(
        body,
        grid=(NBLK,),
        in_specs=[
            pl.BlockSpec((1, BS, D), lambda i: (lin(i) % B, lin(i) // B, 0)),
            pl.BlockSpec((BS, D), lambda i: (lin(i) // B, 0)),
        ],
        out_specs=pl.BlockSpec(
            (1, BS, D), lambda i: (lin(i) % B, lin(i) // B, 0)
        ),
        out_shape=jax.ShapeDtypeStruct((B, S, D), inputs.dtype),
    )(inputs, table)


def kernel(inputs, table):
    sc_out = _sc_part(inputs, table)
    tc_full = _tc_part(inputs, table)
    return lax.dynamic_update_slice(tc_full, sc_out, (0, 0, 0))


# speedup vs baseline: 1.0623x; 1.0623x over previous
import functools

import jax
import jax.numpy as jnp
from jax import lax
from jax.experimental import pallas as pl
from jax.experimental.pallas import tpu as pltpu
from jax.experimental.pallas import tpu_sc as plsc

S_SC = 2048
BS = 2048


def _sc_part(inputs, table):
    B, S, D = inputs.shape
    info = plsc.get_sparse_core_info()
    NC, NS, L = info.num_cores, info.num_subcores, info.num_lanes
    NW = NC * NS
    SPW = S_SC // NW
    CS = 32
    NCHUNK = SPW // CS
    NVC = D // L

    mesh = plsc.VectorSubcoreMesh(core_axis_name="c", subcore_axis_name="s")

    @functools.partial(
        pl.kernel,
        mesh=mesh,
        out_type=jax.ShapeDtypeStruct((1, S_SC, D), jnp.float32),
        scratch_types=[
            pltpu.VMEM((NCHUNK, CS, D), jnp.float32),
            pltpu.VMEM((NCHUNK, CS, D), jnp.float32),
            pltpu.SemaphoreType.DMA((NCHUNK,)),
            pltpu.SemaphoreType.DMA((NCHUNK,)),
            pltpu.SemaphoreType.DMA((NCHUNK,)),
        ],
    )
    def k(x_hbm, t_hbm, o_hbm, tbuf, xbuf, tsem, xsem, osem):
        wid = lax.axis_index("s") * NC + lax.axis_index("c")
        base = wid * SPW

        loads = []
        for c in range(NCHUNK):
            s0 = base + c * CS
            lt = pltpu.async_copy(t_hbm.at[pl.ds(s0, CS)], tbuf.at[c], tsem.at[c])
            lx = pltpu.async_copy(x_hbm.at[0, pl.ds(s0, CS)], xbuf.at[c], xsem.at[c])
            loads.append((lt, lx))

        stores = []
        for c in range(NCHUNK):
            s0 = base + c * CS
            lt, lx = loads[c]
            lt.wait()
            lx.wait()

            def row_body(r, carry, c=c):
                for cc in range(NVC):
                    sl = pl.ds(cc * L, L)
                    xbuf[c, r, sl] = xbuf[c, r, sl] + tbuf[c, r, sl]
                return carry

            lax.fori_loop(0, CS, row_body, 0)
            stores.append(
                pltpu.async_copy(xbuf.at[c], o_hbm.at[0, pl.ds(s0, CS)], osem.at[c])
            )
        for st in stores:
            st.wait()

    return k(inputs, table)


def _tc_part(inputs, table):
    B, S, D = inputs.shape
    NSB = S // BS
    NBLK = B * NSB - S_SC // BS

    def body(x_ref, t_ref, o_ref):
        o_ref[...] = x_ref[...] + t_ref[...]

    skip = S_SC // BS

    def lin(i):
        j = i + 1
        for k in range(1, skip):
            j = j + jnp.where(i >= k * B - 1, 1, 0)
        return j

    return pl.pallas_call(
        body,
        grid=(NBLK,),
        in_specs=[
            pl.BlockSpec((1, BS, D), lambda i: (lin(i) % B, lin(i) // B, 0)),
            pl.BlockSpec((BS, D), lambda i: (lin(i) // B, 0)),
        ],
        out_specs=pl.BlockSpec(
            (1, BS, D), lambda i: (lin(i) % B, lin(i) // B, 0)
        ),
        out_shape=jax.ShapeDtypeStruct((B, S, D), inputs.dtype),
    )(inputs, table)


def kernel(inputs, table):
    sc_out = _sc_part(inputs, table)
    tc_full = _tc_part(inputs, table)
    return lax.dynamic_update_slice(tc_full, sc_out, (0, 0, 0))


# speedup vs baseline: 1.0686x; 1.0060x over previous
import functools

import jax
import jax.numpy as jnp
from jax import lax
from jax.experimental import pallas as pl
from jax.experimental.pallas import tpu as pltpu
from jax.experimental.pallas import tpu_sc as plsc

S_SC = 2048
BS = 2048


def _sc_part(inputs, table):
    B, S, D = inputs.shape
    info = plsc.get_sparse_core_info()
    NC, NS, L = info.num_cores, info.num_subcores, info.num_lanes
    NW = NC * NS
    SPW = S_SC // NW
    CS = 64
    NCHUNK = SPW // CS

    NVC = D // L

    mesh = plsc.VectorSubcoreMesh(core_axis_name="c", subcore_axis_name="s")

    @functools.partial(
        pl.kernel,
        mesh=mesh,
        out_type=jax.ShapeDtypeStruct((1, S_SC, D), jnp.float32),
        scratch_types=[
            pltpu.VMEM((NCHUNK, CS, D), jnp.float32),
            pltpu.VMEM((NCHUNK, CS, D), jnp.float32),
            pltpu.SemaphoreType.DMA((NCHUNK,)),
            pltpu.SemaphoreType.DMA((NCHUNK,)),
            pltpu.SemaphoreType.DMA((NCHUNK,)),
        ],
    )
    def k(x_hbm, t_hbm, o_hbm, tbuf, xbuf, tsem, xsem, osem):
        wid = lax.axis_index("s") * NC + lax.axis_index("c")
        base = wid * SPW

        loads = []
        for c in range(NCHUNK):
            s0 = base + c * CS
            lt = pltpu.async_copy(t_hbm.at[pl.ds(s0, CS)], tbuf.at[c], tsem.at[c])
            lx = pltpu.async_copy(x_hbm.at[0, pl.ds(s0, CS)], xbuf.at[c], xsem.at[c])
            loads.append((lt, lx))

        stores = []
        for c in range(NCHUNK):
            s0 = base + c * CS
            lt, lx = loads[c]
            lt.wait()
            lx.wait()

            def row_body(r, carry, c=c):
                for cc in range(NVC):
                    sl = pl.ds(cc * L, L)
                    xbuf[c, r, sl] = xbuf[c, r, sl] + tbuf[c, r, sl]
                return carry

            lax.fori_loop(0, CS, row_body, 0)
            stores.append(
                pltpu.async_copy(xbuf.at[c], o_hbm.at[0, pl.ds(s0, CS)], osem.at[c])
            )
        for st in stores:
            st.wait()

    return k(inputs, table)


def _tc_part(inputs, table):
    B, S, D = inputs.shape
    NSB = S // BS
    NBLK = B * NSB - S_SC // BS

    def body(x_ref, t_ref, o_ref):
        o_ref[...] = x_ref[...] + t_ref[...]

    skip = S_SC // BS

    def lin(i):
        j = i + 1
        for k in range(1, skip):
            j = j + jnp.where(i >= k * B - 1, 1, 0)
        return j

    return pl.pallas_call(
        body,
        grid=(NBLK,),
        in_specs=[
            pl.BlockSpec((1, BS, D), lambda i: (lin(i) % B, lin(i) // B, 0)),
            pl.BlockSpec((BS, D), lambda i: (lin(i) // B, 0)),
        ],
        out_specs=pl.BlockSpec(
            (1, BS, D), lambda i: (lin(i) % B, lin(i) // B, 0)
        ),
        out_shape=jax.ShapeDtypeStruct((B, S, D), inputs.dtype),
    )(inputs, table)


def kernel(inputs, table):
    sc_out = _sc_part(inputs, table)
    tc_full = _tc_part(inputs, table)
    return lax.dynamic_update_slice(tc_full, sc_out, (0, 0, 0))
